# baseline (speedup 1.0000x reference)
import functools

import jax
import jax.numpy as jnp
from jax import lax
from jax.experimental import pallas as pl
from jax.experimental.pallas import tpu as pltpu
from jax.experimental.pallas import tpu_sc as plsc

NX = 256
NY = 256
B_STATIC = 4
C = 64
SEGS = B_STATIC * NY * NX

NC = 2
NS = 16
NW = NC * NS

N = 200000
N_PAD = 200192
OUT_PAD = N_PAD + 16
NC_H = N_PAD // NC
CH1 = NC_H // NS
NBKT = 256
SPB = SEGS // NBKT
PASSES = NBKT // NW
Q1 = CH1 // 128 + 1
PERM_LEN = NC_H + 1024
STW = 272

_params = pltpu.CompilerParams(use_tc_tiling_on_sc=False, needs_layout_passes=False)
_mesh = plsc.VectorSubcoreMesh(core_axis_name="c", subcore_axis_name="s")


def _route(seg_pad):
    @functools.partial(
        pl.kernel,
        mesh=_mesh,
        compiler_params=_params,
        out_type=(
            jax.ShapeDtypeStruct((PERM_LEN,), jnp.int32),
            jax.ShapeDtypeStruct((PERM_LEN,), jnp.int32),
            jax.ShapeDtypeStruct((NC, STW), jnp.int32),
        ),
        scratch_types=[
            pltpu.VMEM((CH1,), jnp.int32),
            pltpu.VMEM((NBKT,), jnp.int32),
            pltpu.VMEM((NS, NBKT), jnp.int32),
            pltpu.VMEM((NBKT,), jnp.int32),
            pltpu.VMEM((STW,), jnp.int32),
            pltpu.VMEM((Q1 * 128,), jnp.int32),
            pltpu.VMEM((Q1 * 128,), jnp.int32),
            pltpu.VMEM_SHARED((NS, NBKT), jnp.int32),
            pltpu.VMEM_SHARED((PERM_LEN,), jnp.int32),
            pltpu.SemaphoreType.DMA,
        ],
    )
    def k(seg_hbm, perm0, perm1, starts_hbm, seg_v, hist_v, allh_v, cnt_v,
          cst_v, pos_v, ids_v, shared, sperm, sem):
        cid = lax.axis_index("c")
        sid = lax.axis_index("s")
        base = cid * NC_H + sid * CH1

        pltpu.sync_copy(seg_hbm.at[pl.ds(base, CH1)], seg_v)

        zeros16 = jnp.zeros((16,), jnp.int32)
        ones16 = jnp.ones((16,), jnp.int32)
        for j in range(NBKT // 16):
            hist_v[pl.ds(16 * j, 16)] = zeros16

        def hist_body(j, _):
            s = seg_v[pl.ds(j * 16, 16)]
            b = lax.shift_right_logical(s, 10)
            plsc.addupdate_scatter(hist_v, [b], ones16)
            return 0

        lax.fori_loop(0, CH1 // 16, hist_body, 0)

        pltpu.sync_copy(hist_v, shared.at[sid])
        plsc.subcore_barrier()
        pltpu.sync_copy(shared, allh_v)

        carry = jnp.int32(0)
        for j in range(NBKT // 16):
            tot = zeros16
            mine = zeros16
            for t in range(NS):
                h = allh_v[t, pl.ds(16 * j, 16)]
                tot = tot + h
                m = jnp.broadcast_to(sid, (16,)) > t
                mine = mine + jnp.where(m, h, zeros16)
            c = plsc.cumsum(tot)
            excl = c - tot + jnp.broadcast_to(carry, (16,))
            cst_v[pl.ds(16 * j, 16)] = excl
            cnt_v[pl.ds(16 * j, 16)] = excl + mine
            carry = carry + jnp.sum(tot)

        iota16 = lax.broadcasted_iota(jnp.int32, (16,), 0)
        tail_vec = jnp.full((16,), NBKT, jnp.int32)
        cst_v[pl.ds(NBKT, 16)] = jnp.where(iota16 == 0, carry, 0)

        @pl.when(sid == 0)
        def _():
            pltpu.sync_copy(cst_v, starts_hbm.at[cid])

        pos_v[pl.ds(CH1, 16)] = NC_H + iota16
        ids_v[pl.ds(CH1, 16)] = jnp.zeros((16,), jnp.int32)

        def rank_body(j, _):
            sv = seg_v[pl.ds(j * 16, 16)]
            bv = lax.shift_right_logical(sv, 10)
            cur = plsc.load_gather(cnt_v, [bv])
            dup, _last = plsc.scan_count(bv)
            off_vec = cur + dup - 1
            plsc.addupdate_scatter(cnt_v, [bv], ones16)
            o = pl.multiple_of(j * 16, 16)
            gi = base + j * 16 + iota16
            pid = jnp.where(gi >= N, gi - N, gi)
            pos_v[pl.ds(o, 16)] = off_vec
            ids_v[pl.ds(o, 16)] = pid
            return 0

        lax.fori_loop(0, CH1 // 16, rank_body, 0)

        pltpu.async_copy(ids_v, sperm.at[pos_v], sem).wait()
        plsc.subcore_barrier()
        stripe = PERM_LEN // NS

        @pl.when(cid == 0)
        def _():
            pltpu.sync_copy(sperm.at[pl.ds(sid * stripe, stripe)],
                            perm0.at[pl.ds(sid * stripe, stripe)])

        @pl.when(cid == 1)
        def _():
            pltpu.sync_copy(sperm.at[pl.ds(sid * stripe, stripe)],
                            perm1.at[pl.ds(sid * stripe, stripe)])

    return k(seg_pad)


def _scatter_max(feat, seg_pad, perm0, perm1, starts):
    @functools.partial(
        pl.kernel,
        mesh=_mesh,
        compiler_params=_params,
        out_type=jax.ShapeDtypeStruct((OUT_PAD, C), jnp.float32),
        scratch_types=[
            pltpu.VMEM((SPB + 1, C), jnp.float32),
            pltpu.VMEM((NC, STW), jnp.int32),
            pltpu.SMEM((NC * STW,), jnp.int32),
            pltpu.VMEM((256,), jnp.int32),
            pltpu.VMEM((256,), jnp.int32),
            pltpu.VMEM((256,), jnp.int32),
            pltpu.VMEM((256,), jnp.int32),
            pltpu.VMEM((256,), jnp.int32),
            pltpu.VMEM((256,), jnp.int32),
            pltpu.VMEM((256,), jnp.int32),
            pltpu.VMEM((256, C), jnp.float32),
            pltpu.VMEM((256, C), jnp.float32),
            pltpu.VMEM_SHARED((N_PAD,), jnp.int32),
            pltpu.SemaphoreType.DMA,
            pltpu.SemaphoreType.DMA,
            pltpu.SemaphoreType.DMA,
            pltpu.SemaphoreType.DMA,
        ],
    )
    def k(feat_hbm, seg_hbm, perm0, perm1, starts_hbm, out_hbm,
          tab_v, st_v, st_s, idsA, idsB, idcA, idcB, sgvA, sgvB, idoA,
          rowsA, rowsB, sseg, semA1, semA2, semB1, semB2):
        cid = lax.axis_index("c")
        sid = lax.axis_index("s")
        wid = sid * NC + cid

        sstripe = N_PAD // NS
        pltpu.sync_copy(seg_hbm.at[pl.ds(sid * sstripe, sstripe)],
                        sseg.at[pl.ds(sid * sstripe, sstripe)])
        pltpu.sync_copy(starts_hbm, st_v)
        for core in range(NC):
            for j in range(STW // 16):
                v = st_v[core, pl.ds(16 * j, 16)]
                for l in range(16):
                    st_s[core * STW + 16 * j + l] = v[l]

        plsc.subcore_barrier()
        zeros16 = jnp.zeros((16,), jnp.float32)
        iota16 = lax.broadcasted_iota(jnp.int32, (16,), 0)

        def clamp_ids(ids_v, idc_v):
            for cc in range(16):
                idr = ids_v[pl.ds(16 * cc, 16)]
                idc = lax.max(lax.min(idr, jnp.int32(N - 1)), jnp.int32(0))
                idc_v[pl.ds(16 * cc, 16)] = idc

        def rmw(sgv_v, rows_v, lane0, mrel, seg_base):
            def rmw_body(q, _):
                qb = pl.multiple_of(q * 16, 16)
                sg = sgv_v[pl.ds(qb, 16)]
                lane = iota16 + (lane0 + q * 16)
                valid = (lane >= 0) & (lane < mrel)
                sl = sg - seg_base
                sl = jnp.where(valid, sl, jnp.int32(SPB))
                sl = lax.max(lax.min(sl, jnp.int32(SPB)), jnp.int32(0))
                for l in range(16):
                    r = sl[l]
                    for cc in range(C // 16):
                        t = tab_v[r, pl.ds(16 * cc, 16)]
                        v = rows_v[qb + l, pl.ds(16 * cc, 16)]
                        tab_v[r, pl.ds(16 * cc, 16)] = lax.max(t, v)
                return 0

            lax.fori_loop(0, 16, rmw_body, 0)

        def pass_body(p, _):
            g = p * NW + wid
            seg_base = g * SPB

            def zero_body(r, _):
                for cc in range(C // 16):
                    tab_v[r, pl.ds(16 * cc, 16)] = zeros16
                return 0

            lax.fori_loop(0, SPB + 1, zero_body, 0)

            for core in range(NC):
                perm = (perm0, perm1)[core]
                st = st_s[core * STW + g]
                en = st_s[core * STW + g + 1]
                a = lax.bitwise_and(st, jnp.int32(~7))
                head = st - a
                mrel = en - a - head
                nch = lax.div(en - a + 255, 256)

                def pair_body(q, _, perm=perm, a=a, head=head, mrel=mrel,
                              nch=nch, seg_base=seg_base):
                    offA = pl.multiple_of(a + q * 512, 8)
                    offB = pl.multiple_of(a + q * 512 + 256, 8)
                    pltpu.sync_copy(perm.at[pl.ds(offA, 256)], idsA)
                    clamp_ids(idsA, idcA)
                    cpA1 = pltpu.async_copy(feat_hbm.at[idcA], rowsA, semA1)
                    cpA2 = pltpu.async_copy(sseg.at[idcA], sgvA, semA2)
                    has_b = 2 * q + 1 < nch

                    @pl.when(has_b)
                    def _():
                        pltpu.sync_copy(perm.at[pl.ds(offB, 256)], idsB)
                        clamp_ids(idsB, idcB)
                        cpB1 = pltpu.async_copy(feat_hbm.at[idcB], rowsB, semB1)
                        cpB2 = pltpu.async_copy(sseg.at[idcB], sgvB, semB2)

                    cpA1.wait()
                    cpA2.wait()
                    rmw(sgvA, rowsA, 2 * q * 256 - head, mrel, seg_base)

                    @pl.when(has_b)
                    def _():
                        pltpu.make_async_copy(feat_hbm.at[idcB], rowsB,
                                              semB1).wait()
                        pltpu.make_async_copy(sseg.at[idcB], sgvB,
                                              semB2).wait()
                        rmw(sgvB, rowsB, (2 * q + 1) * 256 - head,
                            mrel, seg_base)

                    return 0

                lax.fori_loop(0, lax.div(nch + 1, 2), pair_body, 0)

            for core in range(NC):
                perm = (perm0, perm1)[core]
                st = st_s[core * STW + g]
                en = st_s[core * STW + g + 1]
                a = lax.bitwise_and(st, jnp.int32(~7))
                head = st - a
                mrel = en - a - head
                nch = lax.div(en - a + 255, 256)

                def out_body(kk, _, perm=perm, a=a, head=head, mrel=mrel,
                             seg_base=seg_base):
                    off = pl.multiple_of(a + kk * 512, 8)
                    del off
                    offk = pl.multiple_of(a + kk * 256, 8)
                    pltpu.sync_copy(perm.at[pl.ds(offk, 256)], idsA)
                    clamp_ids(idsA, idcA)
                    pltpu.async_copy(sseg.at[idcA], sgvA, semA2).wait()
                    lane0 = kk * 256 - head

                    def fill_body(q, _):
                        qb = pl.multiple_of(q * 16, 16)
                        sg = sgvA[pl.ds(qb, 16)]
                        lane = iota16 + (lane0 + q * 16)
                        valid = (lane >= 0) & (lane < mrel)
                        sl = sg - seg_base
                        sl = jnp.where(valid, sl, jnp.int32(SPB))
                        sl = lax.max(lax.min(sl, jnp.int32(SPB)),
                                     jnp.int32(0))
                        oid = jnp.where(valid, idcA[pl.ds(qb, 16)],
                                        N_PAD + iota16)
                        idoA[pl.ds(qb, 16)] = oid
                        for l in range(16):
                            r = sl[l]
                            for cc in range(C // 16):
                                rowsA[qb + l, pl.ds(16 * cc, 16)] = (
                                    tab_v[r, pl.ds(16 * cc, 16)])
                        return 0

                    lax.fori_loop(0, 16, fill_body, 0)
                    pltpu.async_copy(rowsA, out_hbm.at[idoA], semA1).wait()
                    return 0

                lax.fori_loop(0, nch, out_body, 0)
            return 0

        lax.fori_loop(0, PASSES, pass_body, 0)

    return k(feat, seg_pad, perm0, perm1, starts)


GCH = 272


def _gather(table, idx):
    n = idx.shape[0]
    n_w = n // NW
    n_iter = n_w // GCH
    assert n_w % GCH == 0

    @functools.partial(
        pl.kernel,
        mesh=_mesh,
        compiler_params=_params,
        out_type=jax.ShapeDtypeStruct((n, C), jnp.float32),
        scratch_types=[
            pltpu.VMEM((GCH,), jnp.int32),
            pltpu.VMEM((GCH, C), jnp.float32),
            pltpu.SemaphoreType.DMA,
        ],
    )
    def k(table_hbm, idx_hbm, out_hbm, idx_v, rows_v, sem):
        wid = lax.axis_index("s") * NC + lax.axis_index("c")
        base = wid * n_w
        for i in range(n_iter):
            off = base + i * GCH
            pltpu.sync_copy(idx_hbm.at[pl.ds(off, GCH)], idx_v)
            pltpu.async_copy(table_hbm.at[idx_v], rows_v, sem).wait()
            pltpu.sync_copy(rows_v, out_hbm.at[pl.ds(off, GCH)])

    return k(table, idx)


def kernel(point_features, pts_coors, batch_size, stride):
    n = point_features.shape[0]
    coors = pts_coors.at[:, 1:3].set(pts_coors[:, 1:3] // stride)
    seg = (coors[:, 0] % batch_size) * (NY * NX) + coors[:, 1] * NX + coors[:, 2]
    seg = seg.astype(jnp.int32)
    seg_pad = jnp.concatenate([seg, seg[: N_PAD - n]])

    perm0, perm1, starts = _route(seg_pad)
    out = _scatter_max(point_features, seg_pad, perm0, perm1, starts)
    return out[:n]


# speedup vs baseline: 1.0227x; 1.0227x over previous
import functools

import jax
import jax.numpy as jnp
from jax import lax
from jax.experimental import pallas as pl
from jax.experimental.pallas import tpu as pltpu
from jax.experimental.pallas import tpu_sc as plsc

NX = 256
NY = 256
B_STATIC = 4
C = 64
SEGS = B_STATIC * NY * NX

NC = 2
NS = 16
NW = NC * NS

N = 200000
N_PAD = 200192
OUT_PAD = N_PAD + 16
NC_H = N_PAD // NC
CH1 = NC_H // NS
NBKT = 256
SPB = SEGS // NBKT
PASSES = NBKT // NW
Q1 = CH1 // 128 + 1
PERM_LEN = NC_H + 1024
STW = 272

_params = pltpu.CompilerParams(use_tc_tiling_on_sc=False, needs_layout_passes=False)
_mesh = plsc.VectorSubcoreMesh(core_axis_name="c", subcore_axis_name="s")


def _route(seg_pad):
    @functools.partial(
        pl.kernel,
        mesh=_mesh,
        compiler_params=_params,
        out_type=(
            jax.ShapeDtypeStruct((PERM_LEN,), jnp.int32),
            jax.ShapeDtypeStruct((PERM_LEN,), jnp.int32),
            jax.ShapeDtypeStruct((NC, STW), jnp.int32),
        ),
        scratch_types=[
            pltpu.VMEM((CH1,), jnp.int32),
            pltpu.VMEM((NBKT,), jnp.int32),
            pltpu.VMEM((NS, NBKT), jnp.int32),
            pltpu.VMEM((NBKT,), jnp.int32),
            pltpu.VMEM((STW,), jnp.int32),
            pltpu.VMEM((Q1 * 128,), jnp.int32),
            pltpu.VMEM((Q1 * 128,), jnp.int32),
            pltpu.VMEM_SHARED((NS, NBKT), jnp.int32),
            pltpu.VMEM_SHARED((PERM_LEN,), jnp.int32),
            pltpu.SemaphoreType.DMA,
        ],
    )
    def k(seg_hbm, perm0, perm1, starts_hbm, seg_v, hist_v, allh_v, cnt_v,
          cst_v, pos_v, ids_v, shared, sperm, sem):
        cid = lax.axis_index("c")
        sid = lax.axis_index("s")
        base = cid * NC_H + sid * CH1

        pltpu.sync_copy(seg_hbm.at[pl.ds(base, CH1)], seg_v)

        zeros16 = jnp.zeros((16,), jnp.int32)
        ones16 = jnp.ones((16,), jnp.int32)
        for j in range(NBKT // 16):
            hist_v[pl.ds(16 * j, 16)] = zeros16

        def hist_body(j, _):
            s = seg_v[pl.ds(j * 16, 16)]
            b = lax.shift_right_logical(s, 10)
            plsc.addupdate_scatter(hist_v, [b], ones16)
            return 0

        lax.fori_loop(0, CH1 // 16, hist_body, 0)

        pltpu.sync_copy(hist_v, shared.at[sid])
        plsc.subcore_barrier()
        pltpu.sync_copy(shared, allh_v)

        carry = jnp.int32(0)
        for j in range(NBKT // 16):
            tot = zeros16
            mine = zeros16
            for t in range(NS):
                h = allh_v[t, pl.ds(16 * j, 16)]
                tot = tot + h
                m = jnp.broadcast_to(sid, (16,)) > t
                mine = mine + jnp.where(m, h, zeros16)
            c = plsc.cumsum(tot)
            excl = c - tot + jnp.broadcast_to(carry, (16,))
            cst_v[pl.ds(16 * j, 16)] = excl
            cnt_v[pl.ds(16 * j, 16)] = excl + mine
            carry = carry + jnp.sum(tot)

        iota16 = lax.broadcasted_iota(jnp.int32, (16,), 0)
        tail_vec = jnp.full((16,), NBKT, jnp.int32)
        cst_v[pl.ds(NBKT, 16)] = jnp.where(iota16 == 0, carry, 0)

        @pl.when(sid == 0)
        def _():
            pltpu.sync_copy(cst_v, starts_hbm.at[cid])

        pos_v[pl.ds(CH1, 16)] = NC_H + iota16
        ids_v[pl.ds(CH1, 16)] = jnp.zeros((16,), jnp.int32)

        def rank_body(j, _):
            sv = seg_v[pl.ds(j * 16, 16)]
            bv = lax.shift_right_logical(sv, 10)
            cur = plsc.load_gather(cnt_v, [bv])
            dup, _last = plsc.scan_count(bv)
            off_vec = cur + dup - 1
            plsc.addupdate_scatter(cnt_v, [bv], ones16)
            o = pl.multiple_of(j * 16, 16)
            gi = base + j * 16 + iota16
            pid = jnp.where(gi >= N, gi - N, gi)
            pos_v[pl.ds(o, 16)] = off_vec
            ids_v[pl.ds(o, 16)] = pid
            return 0

        lax.fori_loop(0, CH1 // 16, rank_body, 0)

        pltpu.async_copy(ids_v, sperm.at[pos_v], sem).wait()
        plsc.subcore_barrier()
        stripe = PERM_LEN // NS

        @pl.when(cid == 0)
        def _():
            pltpu.sync_copy(sperm.at[pl.ds(sid * stripe, stripe)],
                            perm0.at[pl.ds(sid * stripe, stripe)])

        @pl.when(cid == 1)
        def _():
            pltpu.sync_copy(sperm.at[pl.ds(sid * stripe, stripe)],
                            perm1.at[pl.ds(sid * stripe, stripe)])

    return k(seg_pad)


def _scatter_max(feat, seg_pad, perm0, perm1, starts):
    @functools.partial(
        pl.kernel,
        mesh=_mesh,
        compiler_params=_params,
        out_type=jax.ShapeDtypeStruct((OUT_PAD, C), jnp.float32),
        scratch_types=[
            pltpu.VMEM((SPB + 1, C), jnp.float32),
            pltpu.VMEM((NC, STW), jnp.int32),
            pltpu.SMEM((NC * STW,), jnp.int32),
            pltpu.VMEM((256,), jnp.int32),
            pltpu.VMEM((256,), jnp.int32),
            pltpu.VMEM((256,), jnp.int32),
            pltpu.VMEM((256,), jnp.int32),
            pltpu.VMEM((256,), jnp.int32),
            pltpu.VMEM((256,), jnp.int32),
            pltpu.VMEM((256,), jnp.int32),
            pltpu.VMEM((256,), jnp.int32),
            pltpu.VMEM((256, C), jnp.float32),
            pltpu.VMEM((256, C), jnp.float32),
            pltpu.VMEM_SHARED((N_PAD,), jnp.int32),
            pltpu.SemaphoreType.DMA,
            pltpu.SemaphoreType.DMA,
            pltpu.SemaphoreType.DMA,
            pltpu.SemaphoreType.DMA,
        ],
    )
    def k(feat_hbm, seg_hbm, perm0, perm1, starts_hbm, out_hbm,
          tab_v, st_v, st_s, idsA, idsB, idcA, idcB, sgvA, sgvB, idoA, idoB,
          rowsA, rowsB, sseg, semA1, semA2, semB1, semB2):
        cid = lax.axis_index("c")
        sid = lax.axis_index("s")
        wid = sid * NC + cid

        sstripe = N_PAD // NS
        pltpu.sync_copy(seg_hbm.at[pl.ds(sid * sstripe, sstripe)],
                        sseg.at[pl.ds(sid * sstripe, sstripe)])
        pltpu.sync_copy(starts_hbm, st_v)
        for core in range(NC):
            for j in range(STW // 16):
                v = st_v[core, pl.ds(16 * j, 16)]
                for l in range(16):
                    st_s[core * STW + 16 * j + l] = v[l]

        plsc.subcore_barrier()
        zeros16 = jnp.zeros((16,), jnp.float32)
        iota16 = lax.broadcasted_iota(jnp.int32, (16,), 0)

        def clamp_ids(ids_v, idc_v):
            for cc in range(16):
                idr = ids_v[pl.ds(16 * cc, 16)]
                idc = lax.max(lax.min(idr, jnp.int32(N - 1)), jnp.int32(0))
                idc_v[pl.ds(16 * cc, 16)] = idc

        def rmw(sgv_v, rows_v, lane0, mrel, seg_base):
            def rmw_body(q, _):
                qb = pl.multiple_of(q * 16, 16)
                sg = sgv_v[pl.ds(qb, 16)]
                lane = iota16 + (lane0 + q * 16)
                valid = (lane >= 0) & (lane < mrel)
                sl = sg - seg_base
                sl = jnp.where(valid, sl, jnp.int32(SPB))
                sl = lax.max(lax.min(sl, jnp.int32(SPB)), jnp.int32(0))
                for l in range(16):
                    r = sl[l]
                    for cc in range(C // 16):
                        t = tab_v[r, pl.ds(16 * cc, 16)]
                        v = rows_v[qb + l, pl.ds(16 * cc, 16)]
                        tab_v[r, pl.ds(16 * cc, 16)] = lax.max(t, v)
                return 0

            lax.fori_loop(0, 16, rmw_body, 0)

        def pass_body(p, _):
            g = p * NW + wid
            seg_base = g * SPB

            def zero_body(r, _):
                for cc in range(C // 16):
                    tab_v[r, pl.ds(16 * cc, 16)] = zeros16
                return 0

            lax.fori_loop(0, SPB + 1, zero_body, 0)

            for core in range(NC):
                perm = (perm0, perm1)[core]
                st = st_s[core * STW + g]
                en = st_s[core * STW + g + 1]
                a = lax.bitwise_and(st, jnp.int32(~7))
                head = st - a
                mrel = en - a - head
                nch = lax.div(en - a + 255, 256)

                def pair_body(q, _, perm=perm, a=a, head=head, mrel=mrel,
                              nch=nch, seg_base=seg_base):
                    offA = pl.multiple_of(a + q * 512, 8)
                    offB = pl.multiple_of(a + q * 512 + 256, 8)
                    pltpu.sync_copy(perm.at[pl.ds(offA, 256)], idsA)
                    clamp_ids(idsA, idcA)
                    cpA1 = pltpu.async_copy(feat_hbm.at[idcA], rowsA, semA1)
                    cpA2 = pltpu.async_copy(sseg.at[idcA], sgvA, semA2)
                    has_b = 2 * q + 1 < nch

                    @pl.when(has_b)
                    def _():
                        pltpu.sync_copy(perm.at[pl.ds(offB, 256)], idsB)
                        clamp_ids(idsB, idcB)
                        cpB1 = pltpu.async_copy(feat_hbm.at[idcB], rowsB, semB1)
                        cpB2 = pltpu.async_copy(sseg.at[idcB], sgvB, semB2)

                    cpA1.wait()
                    cpA2.wait()
                    rmw(sgvA, rowsA, 2 * q * 256 - head, mrel, seg_base)

                    @pl.when(has_b)
                    def _():
                        pltpu.make_async_copy(feat_hbm.at[idcB], rowsB,
                                              semB1).wait()
                        pltpu.make_async_copy(sseg.at[idcB], sgvB,
                                              semB2).wait()
                        rmw(sgvB, rowsB, (2 * q + 1) * 256 - head,
                            mrel, seg_base)

                    return 0

                lax.fori_loop(0, lax.div(nch + 1, 2), pair_body, 0)

            def fill_rows(sgv_v, idc_v, ido_v, rows_v, lane0, mrel, seg_base):
                def fill_body(q, _):
                    qb = pl.multiple_of(q * 16, 16)
                    sg = sgv_v[pl.ds(qb, 16)]
                    lane = iota16 + (lane0 + q * 16)
                    valid = (lane >= 0) & (lane < mrel)
                    sl = sg - seg_base
                    sl = jnp.where(valid, sl, jnp.int32(SPB))
                    sl = lax.max(lax.min(sl, jnp.int32(SPB)), jnp.int32(0))
                    oid = jnp.where(valid, idc_v[pl.ds(qb, 16)],
                                    N_PAD + iota16)
                    ido_v[pl.ds(qb, 16)] = oid
                    for l in range(16):
                        r = sl[l]
                        for cc in range(C // 16):
                            rows_v[qb + l, pl.ds(16 * cc, 16)] = (
                                tab_v[r, pl.ds(16 * cc, 16)])
                    return 0

                lax.fori_loop(0, 16, fill_body, 0)

            for core in range(NC):
                perm = (perm0, perm1)[core]
                st = st_s[core * STW + g]
                en = st_s[core * STW + g + 1]
                a = lax.bitwise_and(st, jnp.int32(~7))
                head = st - a
                mrel = en - a - head
                nch = lax.div(en - a + 255, 256)

                def opair_body(q, _, perm=perm, a=a, head=head, mrel=mrel,
                               nch=nch, seg_base=seg_base):
                    offA = pl.multiple_of(a + q * 512, 8)
                    offB = pl.multiple_of(a + q * 512 + 256, 8)
                    pltpu.sync_copy(perm.at[pl.ds(offA, 256)], idsA)
                    clamp_ids(idsA, idcA)
                    cpA2 = pltpu.async_copy(sseg.at[idcA], sgvA, semA2)
                    has_b = 2 * q + 1 < nch

                    @pl.when(has_b)
                    def _():
                        pltpu.sync_copy(perm.at[pl.ds(offB, 256)], idsB)
                        clamp_ids(idsB, idcB)
                        pltpu.async_copy(sseg.at[idcB], sgvB, semB2)

                    cpA2.wait()
                    fill_rows(sgvA, idcA, idoA, rowsA,
                              2 * q * 256 - head, mrel, seg_base)
                    pltpu.async_copy(rowsA, out_hbm.at[idoA], semA1)

                    @pl.when(has_b)
                    def _():
                        pltpu.make_async_copy(sseg.at[idcB], sgvB,
                                              semB2).wait()
                        fill_rows(sgvB, idcB, idoB, rowsB,
                                  (2 * q + 1) * 256 - head, mrel, seg_base)
                        pltpu.async_copy(rowsB, out_hbm.at[idoB], semB1)

                    pltpu.make_async_copy(rowsA, out_hbm.at[idoA],
                                          semA1).wait()

                    @pl.when(has_b)
                    def _():
                        pltpu.make_async_copy(rowsB, out_hbm.at[idoB],
                                              semB1).wait()

                    return 0

                lax.fori_loop(0, lax.div(nch + 1, 2), opair_body, 0)
            return 0

        lax.fori_loop(0, PASSES, pass_body, 0)

    return k(feat, seg_pad, perm0, perm1, starts)


GCH = 272


def _gather(table, idx):
    n = idx.shape[0]
    n_w = n // NW
    n_iter = n_w // GCH
    assert n_w % GCH == 0

    @functools.partial(
        pl.kernel,
        mesh=_mesh,
        compiler_params=_params,
        out_type=jax.ShapeDtypeStruct((n, C), jnp.float32),
        scratch_types=[
            pltpu.VMEM((GCH,), jnp.int32),
            pltpu.VMEM((GCH, C), jnp.float32),
            pltpu.SemaphoreType.DMA,
        ],
    )
    def k(table_hbm, idx_hbm, out_hbm, idx_v, rows_v, sem):
        wid = lax.axis_index("s") * NC + lax.axis_index("c")
        base = wid * n_w
        for i in range(n_iter):
            off = base + i * GCH
            pltpu.sync_copy(idx_hbm.at[pl.ds(off, GCH)], idx_v)
            pltpu.async_copy(table_hbm.at[idx_v], rows_v, sem).wait()
            pltpu.sync_copy(rows_v, out_hbm.at[pl.ds(off, GCH)])

    return k(table, idx)


def kernel(point_features, pts_coors, batch_size, stride):
    n = point_features.shape[0]
    coors = pts_coors.at[:, 1:3].set(pts_coors[:, 1:3] // stride)
    seg = (coors[:, 0] % batch_size) * (NY * NX) + coors[:, 1] * NX + coors[:, 2]
    seg = seg.astype(jnp.int32)
    seg_pad = jnp.concatenate([seg, seg[: N_PAD - n]])

    perm0, perm1, starts = _route(seg_pad)
    out = _scatter_max(point_features, seg_pad, perm0, perm1, starts)
    return out[:n]


# speedup vs baseline: 1.1797x; 1.1535x over previous
import functools

import jax
import jax.numpy as jnp
from jax import lax
from jax.experimental import pallas as pl
from jax.experimental.pallas import tpu as pltpu
from jax.experimental.pallas import tpu_sc as plsc

NX = 256
NY = 256
B_STATIC = 4
C = 64
SEGS = B_STATIC * NY * NX

NC = 2
NS = 16
NW = NC * NS

N = 200000
N_PAD = 200192
NC_H = N_PAD // NC
CH1 = NC_H // NS
NBKT = 256
SPB = SEGS // NBKT
PASSES = NBKT // NW
Q1 = CH1 // 128 + 1
PERM_LEN = NC_H + 1024
STW = 272

_params = pltpu.CompilerParams(use_tc_tiling_on_sc=False, needs_layout_passes=False)
_mesh = plsc.VectorSubcoreMesh(core_axis_name="c", subcore_axis_name="s")


def _route(seg_pad):
    @functools.partial(
        pl.kernel,
        mesh=_mesh,
        compiler_params=_params,
        out_type=(
            jax.ShapeDtypeStruct((PERM_LEN,), jnp.int32),
            jax.ShapeDtypeStruct((PERM_LEN,), jnp.int32),
            jax.ShapeDtypeStruct((NC, STW), jnp.int32),
        ),
        scratch_types=[
            pltpu.VMEM((CH1,), jnp.int32),
            pltpu.VMEM((NBKT,), jnp.int32),
            pltpu.VMEM((NS, NBKT), jnp.int32),
            pltpu.VMEM((NBKT,), jnp.int32),
            pltpu.VMEM((STW,), jnp.int32),
            pltpu.VMEM((Q1 * 128,), jnp.int32),
            pltpu.VMEM((Q1 * 128,), jnp.int32),
            pltpu.VMEM_SHARED((NS, NBKT), jnp.int32),
            pltpu.VMEM_SHARED((PERM_LEN,), jnp.int32),
            pltpu.SemaphoreType.DMA,
        ],
    )
    def k(seg_hbm, perm0, perm1, starts_hbm, seg_v, hist_v, allh_v, cnt_v,
          cst_v, pos_v, ids_v, shared, sperm, sem):
        cid = lax.axis_index("c")
        sid = lax.axis_index("s")
        base = cid * NC_H + sid * CH1

        pltpu.sync_copy(seg_hbm.at[pl.ds(base, CH1)], seg_v)

        zeros16 = jnp.zeros((16,), jnp.int32)
        ones16 = jnp.ones((16,), jnp.int32)
        for j in range(NBKT // 16):
            hist_v[pl.ds(16 * j, 16)] = zeros16

        def hist_body(j, _):
            s = seg_v[pl.ds(j * 16, 16)]
            b = lax.shift_right_logical(s, 10)
            plsc.addupdate_scatter(hist_v, [b], ones16)
            return 0

        lax.fori_loop(0, CH1 // 16, hist_body, 0)

        pltpu.sync_copy(hist_v, shared.at[sid])
        plsc.subcore_barrier()
        pltpu.sync_copy(shared, allh_v)

        carry = jnp.int32(0)
        for j in range(NBKT // 16):
            tot = zeros16
            mine = zeros16
            for t in range(NS):
                h = allh_v[t, pl.ds(16 * j, 16)]
                tot = tot + h
                m = jnp.broadcast_to(sid, (16,)) > t
                mine = mine + jnp.where(m, h, zeros16)
            c = plsc.cumsum(tot)
            excl = c - tot + jnp.broadcast_to(carry, (16,))
            cst_v[pl.ds(16 * j, 16)] = excl
            cnt_v[pl.ds(16 * j, 16)] = excl + mine
            carry = carry + jnp.sum(tot)

        iota16 = lax.broadcasted_iota(jnp.int32, (16,), 0)
        tail_vec = jnp.full((16,), NBKT, jnp.int32)
        cst_v[pl.ds(NBKT, 16)] = jnp.where(iota16 == 0, carry, 0)

        @pl.when(sid == 0)
        def _():
            pltpu.sync_copy(cst_v, starts_hbm.at[cid])

        pos_v[pl.ds(CH1, 16)] = NC_H + iota16
        ids_v[pl.ds(CH1, 16)] = jnp.zeros((16,), jnp.int32)

        def rank_body(j, _):
            sv = seg_v[pl.ds(j * 16, 16)]
            bv = lax.shift_right_logical(sv, 10)
            cur = plsc.load_gather(cnt_v, [bv])
            dup, _last = plsc.scan_count(bv)
            off_vec = cur + dup - 1
            plsc.addupdate_scatter(cnt_v, [bv], ones16)
            o = pl.multiple_of(j * 16, 16)
            gi = base + j * 16 + iota16
            pid = jnp.where(gi >= N, gi - N, gi)
            pos_v[pl.ds(o, 16)] = off_vec
            ids_v[pl.ds(o, 16)] = pid
            return 0

        lax.fori_loop(0, CH1 // 16, rank_body, 0)

        pltpu.async_copy(ids_v, sperm.at[pos_v], sem).wait()
        plsc.subcore_barrier()
        stripe = PERM_LEN // NS

        @pl.when(cid == 0)
        def _():
            pltpu.sync_copy(sperm.at[pl.ds(sid * stripe, stripe)],
                            perm0.at[pl.ds(sid * stripe, stripe)])

        @pl.when(cid == 1)
        def _():
            pltpu.sync_copy(sperm.at[pl.ds(sid * stripe, stripe)],
                            perm1.at[pl.ds(sid * stripe, stripe)])

    return k(seg_pad)


def _scatter_max(feat, seg_pad, perm0, perm1, starts):
    @functools.partial(
        pl.kernel,
        mesh=_mesh,
        compiler_params=_params,
        out_type=jax.ShapeDtypeStruct((SEGS, C), jnp.float32),
        scratch_types=[
            pltpu.VMEM((SPB + 1, C), jnp.float32),
            pltpu.VMEM((NC, STW), jnp.int32),
            pltpu.SMEM((NC * STW,), jnp.int32),
            pltpu.VMEM((256,), jnp.int32),
            pltpu.VMEM((256,), jnp.int32),
            pltpu.VMEM((256,), jnp.int32),
            pltpu.VMEM((256,), jnp.int32),
            pltpu.VMEM((256,), jnp.int32),
            pltpu.VMEM((256,), jnp.int32),
            pltpu.VMEM((256, C), jnp.float32),
            pltpu.VMEM((256, C), jnp.float32),
            pltpu.VMEM_SHARED((N_PAD,), jnp.int32),
            pltpu.SemaphoreType.DMA,
            pltpu.SemaphoreType.DMA,
            pltpu.SemaphoreType.DMA,
            pltpu.SemaphoreType.DMA,
        ],
    )
    def k(feat_hbm, seg_hbm, perm0, perm1, starts_hbm, table_hbm,
          tab_v, st_v, st_s, idsA, idsB, idcA, idcB, sgvA, sgvB,
          rowsA, rowsB, sseg, semA1, semA2, semB1, semB2):
        cid = lax.axis_index("c")
        sid = lax.axis_index("s")
        wid = sid * NC + cid

        sstripe = N_PAD // NS
        pltpu.sync_copy(seg_hbm.at[pl.ds(sid * sstripe, sstripe)],
                        sseg.at[pl.ds(sid * sstripe, sstripe)])
        pltpu.sync_copy(starts_hbm, st_v)
        for core in range(NC):
            for j in range(STW // 16):
                v = st_v[core, pl.ds(16 * j, 16)]
                for l in range(16):
                    st_s[core * STW + 16 * j + l] = v[l]

        plsc.subcore_barrier()
        zeros16 = jnp.zeros((16,), jnp.float32)
        iota16 = lax.broadcasted_iota(jnp.int32, (16,), 0)

        def clamp_ids(ids_v, idc_v):
            for cc in range(16):
                idr = ids_v[pl.ds(16 * cc, 16)]
                idc = lax.max(lax.min(idr, jnp.int32(N - 1)), jnp.int32(0))
                idc_v[pl.ds(16 * cc, 16)] = idc

        def rmw(sgv_v, rows_v, lane0, mrel, seg_base):
            def rmw_body(q, _):
                qb = pl.multiple_of(q * 16, 16)
                sg = sgv_v[pl.ds(qb, 16)]
                lane = iota16 + (lane0 + q * 16)
                valid = (lane >= 0) & (lane < mrel)
                sl = sg - seg_base
                sl = jnp.where(valid, sl, jnp.int32(SPB))
                sl = lax.max(lax.min(sl, jnp.int32(SPB)), jnp.int32(0))
                for l in range(16):
                    r = sl[l]
                    for cc in range(C // 16):
                        t = tab_v[r, pl.ds(16 * cc, 16)]
                        v = rows_v[qb + l, pl.ds(16 * cc, 16)]
                        tab_v[r, pl.ds(16 * cc, 16)] = lax.max(t, v)
                return 0

            lax.fori_loop(0, 16, rmw_body, 0)

        def pass_body(p, _):
            g = p * NW + wid
            seg_base = g * SPB

            def zero_body(r4, _):
                for u in range(4):
                    r = r4 * 4 + u
                    for cc in range(C // 16):
                        tab_v[r, pl.ds(16 * cc, 16)] = zeros16
                return 0

            lax.fori_loop(0, SPB // 4, zero_body, 0)
            for cc in range(C // 16):
                tab_v[SPB, pl.ds(16 * cc, 16)] = zeros16

            for core in range(NC):
                perm = (perm0, perm1)[core]
                st = st_s[core * STW + g]
                en = st_s[core * STW + g + 1]
                a = lax.bitwise_and(st, jnp.int32(~7))
                head = st - a
                mrel = en - a - head
                nch = lax.div(en - a + 255, 256)

                def pair_body(q, _, perm=perm, a=a, head=head, mrel=mrel,
                              nch=nch, seg_base=seg_base):
                    offA = pl.multiple_of(a + q * 512, 8)
                    offB = pl.multiple_of(a + q * 512 + 256, 8)
                    pltpu.sync_copy(perm.at[pl.ds(offA, 256)], idsA)
                    clamp_ids(idsA, idcA)
                    cpA1 = pltpu.async_copy(feat_hbm.at[idcA], rowsA, semA1)
                    cpA2 = pltpu.async_copy(sseg.at[idcA], sgvA, semA2)
                    has_b = 2 * q + 1 < nch

                    @pl.when(has_b)
                    def _():
                        pltpu.sync_copy(perm.at[pl.ds(offB, 256)], idsB)
                        clamp_ids(idsB, idcB)
                        cpB1 = pltpu.async_copy(feat_hbm.at[idcB], rowsB, semB1)
                        cpB2 = pltpu.async_copy(sseg.at[idcB], sgvB, semB2)

                    cpA1.wait()
                    cpA2.wait()
                    rmw(sgvA, rowsA, 2 * q * 256 - head, mrel, seg_base)

                    @pl.when(has_b)
                    def _():
                        pltpu.make_async_copy(feat_hbm.at[idcB], rowsB,
                                              semB1).wait()
                        pltpu.make_async_copy(sseg.at[idcB], sgvB,
                                              semB2).wait()
                        rmw(sgvB, rowsB, (2 * q + 1) * 256 - head,
                            mrel, seg_base)

                    return 0

                lax.fori_loop(0, lax.div(nch + 1, 2), pair_body, 0)

            pltpu.sync_copy(
                tab_v.at[pl.ds(0, SPB)],
                table_hbm.at[pl.ds(pl.multiple_of(seg_base, SPB), SPB)])
            return 0

        lax.fori_loop(0, PASSES, pass_body, 0)

    return k(feat, seg_pad, perm0, perm1, starts)


GCH = 272


def _gather(table, idx):
    n = idx.shape[0]
    n_w = n // NW
    n_iter = n_w // GCH
    assert n_w % GCH == 0

    @functools.partial(
        pl.kernel,
        mesh=_mesh,
        compiler_params=_params,
        out_type=jax.ShapeDtypeStruct((n, C), jnp.float32),
        scratch_types=[
            pltpu.VMEM((GCH,), jnp.int32),
            pltpu.VMEM((GCH,), jnp.int32),
            pltpu.VMEM((GCH, C), jnp.float32),
            pltpu.VMEM((GCH, C), jnp.float32),
            pltpu.SemaphoreType.DMA,
            pltpu.SemaphoreType.DMA,
        ],
    )
    def k(table_hbm, idx_hbm, out_hbm, idxA, idxB, rowsA, rowsB, semA, semB):
        wid = lax.axis_index("s") * NC + lax.axis_index("c")
        base = wid * n_w
        idx_bufs = (idxA, idxB)
        row_bufs = (rowsA, rowsB)
        sems = (semA, semB)
        pltpu.sync_copy(idx_hbm.at[pl.ds(base, GCH)], idxA)
        pltpu.async_copy(table_hbm.at[idxA], rowsA, semA)
        for i in range(n_iter):
            bi = i % 2
            if i + 1 < n_iter:
                off2 = base + (i + 1) * GCH
                nb = (i + 1) % 2
                pltpu.sync_copy(idx_hbm.at[pl.ds(off2, GCH)], idx_bufs[nb])
                pltpu.async_copy(table_hbm.at[idx_bufs[nb]], row_bufs[nb],
                                 sems[nb])
            off = base + i * GCH
            pltpu.make_async_copy(table_hbm.at[idx_bufs[bi]], row_bufs[bi],
                                  sems[bi]).wait()
            pltpu.sync_copy(row_bufs[bi], out_hbm.at[pl.ds(off, GCH)])

    return k(table, idx)


def kernel(point_features, pts_coors, batch_size, stride):
    n = point_features.shape[0]
    coors = pts_coors.at[:, 1:3].set(pts_coors[:, 1:3] // stride)
    seg = (coors[:, 0] % batch_size) * (NY * NX) + coors[:, 1] * NX + coors[:, 2]
    seg = seg.astype(jnp.int32)
    seg_pad = jnp.concatenate([seg, seg[: N_PAD - n]])

    perm0, perm1, starts = _route(seg_pad)
    table = _scatter_max(point_features, seg_pad, perm0, perm1, starts)
    out = _gather(table, seg_pad)
    return out[:n]
